# Initial kernel scaffold; baseline (speedup 1.0000x reference)
import jax
import jax.numpy as jnp
from jax.experimental import pallas as pl


def kernel(queries, memory_bank):
    raise NotImplementedError("write your pallas kernel here")


# speedup vs baseline: 2.6879x; 2.6879x over previous
import functools

import jax
import jax.numpy as jnp
from jax.experimental import pallas as pl
from jax.experimental.pallas import tpu as pltpu

Q = 1024
D = 16
K = 100000
NN = 9
CHUNK = 2048
NSTEPS = (K + CHUNK - 1) // CHUNK
KPAD = NSTEPS * CHUNK
PADV = 1.0e8
BIGV = 3.0e30
BIGI = 1.0e9


def _topk_chunk(d2, iota, n):
    vals, idxs = [], []
    for _ in range(n):
        mv = jnp.min(d2, axis=1, keepdims=True)
        cand = jnp.where(d2 == mv, iota, BIGI)
        mi = jnp.min(cand, axis=1, keepdims=True)
        vals.append(mv)
        idxs.append(mi)
        d2 = jnp.where(cand == mi, BIGV, d2)
    return jnp.concatenate(vals, axis=1), jnp.concatenate(idxs, axis=1)


def _knn_kernel(q_ref, mt_ref, sv_ref, si_ref, rv_ref, ri_ref):
    k = pl.program_id(0)

    @pl.when(k == 0)
    def _init():
        rv_ref[...] = jnp.full((Q, 16), BIGV, jnp.float32)
        ri_ref[...] = jnp.full((Q, 16), BIGI, jnp.float32)

    q = q_ref[...]
    mt = mt_ref[...]
    dot = jnp.dot(q, mt, preferred_element_type=jnp.float32)
    m2 = jnp.sum(mt * mt, axis=0, keepdims=True)
    q2 = jnp.sum(q * q, axis=1, keepdims=True)
    d2 = (q2 + m2) - 2.0 * dot

    base = (k * CHUNK).astype(jnp.float32)
    iota = jax.lax.broadcasted_iota(jnp.int32, (1, CHUNK), 1).astype(
        jnp.float32) + base

    cvals, cidx = _topk_chunk(d2, iota, NN)

    W = jnp.concatenate([rv_ref[...], cvals], axis=1)
    I = jnp.concatenate([ri_ref[...], cidx], axis=1)
    nv, ni = [], []
    for _ in range(NN):
        mv = jnp.min(W, axis=1, keepdims=True)
        cand = jnp.where(W == mv, I, BIGI)
        mi = jnp.min(cand, axis=1, keepdims=True)
        nv.append(mv)
        ni.append(mi)
        W = jnp.where((W == mv) & (I == mi), BIGV, W)
    pad = jnp.full((Q, 16 - NN), BIGV, jnp.float32)
    padi = jnp.full((Q, 16 - NN), BIGI, jnp.float32)
    rv_ref[...] = jnp.concatenate(nv + [pad], axis=1)
    ri_ref[...] = jnp.concatenate(ni + [padi], axis=1)

    @pl.when(k == NSTEPS - 1)
    def _finish():
        sv_ref[...] = jnp.sqrt(jnp.maximum(rv_ref[...], 1e-12))
        si_ref[...] = ri_ref[...]


def kernel(queries, memory_bank):
    mt = jnp.pad(memory_bank, ((0, KPAD - K), (0, 0)),
                 constant_values=PADV).T
    grid = (NSTEPS,)
    sv, si = pl.pallas_call(
        _knn_kernel,
        grid=grid,
        in_specs=[
            pl.BlockSpec((Q, D), lambda k: (0, 0)),
            pl.BlockSpec((D, CHUNK), lambda k: (0, k)),
        ],
        out_specs=[
            pl.BlockSpec((Q, 16), lambda k: (0, 0)),
            pl.BlockSpec((Q, 16), lambda k: (0, 0)),
        ],
        out_shape=[
            jax.ShapeDtypeStruct((Q, 16), jnp.float32),
            jax.ShapeDtypeStruct((Q, 16), jnp.float32),
        ],
        scratch_shapes=[
            pltpu.VMEM((Q, 16), jnp.float32),
            pltpu.VMEM((Q, 16), jnp.float32),
        ],
        compiler_params=pltpu.CompilerParams(
            dimension_semantics=("arbitrary",),
        ),
    )(queries, mt)
    return sv[:, :NN], si[:, :NN].astype(jnp.int32)
